# baseline (speedup 1.0000x reference)
import jax
import jax.numpy as jnp
from jax.experimental import pallas as pl
from jax.experimental.pallas import tpu as pltpu

B = 4
S = 2048
D = 768
P = 1000
TOP_K = 4
BLK = 256
N_BLK = S // BLK
OUT_DEPTH = 3


def _body(x_hbm, prompt_hbm, out_hbm, sim_ref, idx_ref, rsim_ref,
          xbuf, sbuf, pbuf, acc, first4, tbuf, head,
          in_sems, out_sems, p_sem, head_sem, tail_sem):

    def in_copy(blk):
        return pltpu.make_async_copy(
            x_hbm.at[:, pl.ds(blk * BLK, BLK), :], xbuf.at[blk],
            in_sems.at[blk])

    def out_copy(blk):
        return pltpu.make_async_copy(
            sbuf.at[blk % OUT_DEPTH],
            out_hbm.at[:, pl.ds(blk * BLK, BLK), :],
            out_sems.at[blk % OUT_DEPTH])

    for blk in range(N_BLK):
        in_copy(blk).start()
    pltpu.make_async_copy(prompt_hbm, pbuf, p_sem).start()

    acc_v = jnp.zeros((B, D), jnp.float32)
    tail = None
    for blk in range(N_BLK):
        in_copy(blk).wait()
        v = xbuf[blk]
        acc_v = acc_v + jnp.sum(v, axis=1)
        if blk == 0:
            first4[...] = v[:, :TOP_K, :]
            shifted = jnp.concatenate(
                [v[:, :TOP_K, :], v[:, :BLK - TOP_K, :]], axis=1)
        else:
            shifted = jnp.concatenate([tail, v[:, :BLK - TOP_K, :]], axis=1)
        tail = v[:, BLK - TOP_K:, :]
        if blk >= OUT_DEPTH:
            out_copy(blk - OUT_DEPTH).wait()
        sbuf[blk % OUT_DEPTH] = shifted
        out_copy(blk).start()
    acc[...] = acc_v

    mean = acc_v * (1.0 / S)
    xn = mean * jax.lax.rsqrt(
        jnp.maximum(jnp.sum(mean * mean, axis=1, keepdims=True), 1e-12))
    pltpu.make_async_copy(prompt_hbm, pbuf, p_sem).wait()
    p = pbuf[...]
    pn = p * jax.lax.rsqrt(
        jnp.maximum(jnp.sum(p * p, axis=1, keepdims=True), 1e-12))
    sim = jax.lax.dot_general(
        xn, pn, (((1,), (1,)), ((), ())),
        preferred_element_type=jnp.float32)
    sim_ref[...] = sim

    iota = jax.lax.broadcasted_iota(jnp.int32, (B, P), 1)
    s = sim
    total = jnp.float32(0.0)
    idx_cols = []
    bp_cols = []
    for k in range(TOP_K):
        m = jnp.max(s, axis=1, keepdims=True)
        eq = s == m
        ik = jnp.min(jnp.where(eq, iota, P), axis=1)
        sel = iota == ik[:, None]
        idx_cols.append(ik)
        total += jnp.sum(m)
        bp_cols.append(jax.lax.dot_general(
            sel.astype(jnp.float32), p, (((1,), (0,)), ((), ())),
            preferred_element_type=jnp.float32))
        s = jnp.where(sel, -jnp.inf, s)
    idx_ref[...] = jnp.stack(idx_cols, axis=1)
    rsim_ref[...] = jnp.reshape(total * (1.0 / B), (1, 1))

    head[...] = jnp.concatenate(
        [jnp.stack(bp_cols, axis=1), first4[...]], axis=1)
    hcopy = pltpu.make_async_copy(
        head, out_hbm.at[:, pl.ds(0, 2 * TOP_K), :], head_sem)
    hcopy.start()
    tbuf[...] = tail
    tcopy = pltpu.make_async_copy(
        tbuf, out_hbm.at[:, pl.ds(S, TOP_K), :], tail_sem)
    tcopy.start()
    for blk in range(N_BLK - OUT_DEPTH, N_BLK):
        out_copy(blk).wait()
    hcopy.wait()
    tcopy.wait()


def kernel(x_embed, prompt):
    out_shapes = (
        jax.ShapeDtypeStruct((B, TOP_K + S, D), jnp.float32),
        jax.ShapeDtypeStruct((B, P), jnp.float32),
        jax.ShapeDtypeStruct((B, TOP_K), jnp.int32),
        jax.ShapeDtypeStruct((1, 1), jnp.float32),
    )
    prompted, sim, idx, rsim = pl.pallas_call(
        _body,
        in_specs=[
            pl.BlockSpec(memory_space=pl.MemorySpace.ANY),
            pl.BlockSpec(memory_space=pl.MemorySpace.ANY),
        ],
        out_specs=(
            pl.BlockSpec(memory_space=pl.MemorySpace.ANY),
            pl.BlockSpec((B, P), lambda: (0, 0)),
            pl.BlockSpec((B, TOP_K), lambda: (0, 0)),
            pl.BlockSpec((1, 1), lambda: (0, 0)),
        ),
        out_shape=out_shapes,
        scratch_shapes=[
            pltpu.VMEM((N_BLK, B, BLK, D), jnp.float32),
            pltpu.VMEM((OUT_DEPTH, B, BLK, D), jnp.float32),
            pltpu.VMEM((P, D), jnp.float32),
            pltpu.VMEM((B, D), jnp.float32),
            pltpu.VMEM((B, TOP_K, D), jnp.float32),
            pltpu.VMEM((B, TOP_K, D), jnp.float32),
            pltpu.VMEM((B, 2 * TOP_K, D), jnp.float32),
            pltpu.SemaphoreType.DMA((N_BLK,)),
            pltpu.SemaphoreType.DMA((OUT_DEPTH,)),
            pltpu.SemaphoreType.DMA,
            pltpu.SemaphoreType.DMA,
            pltpu.SemaphoreType.DMA,
        ],
    )(x_embed, prompt)
    return prompted, rsim[0, 0], sim, idx


# speedup vs baseline: 1.0040x; 1.0040x over previous
import jax
import jax.numpy as jnp
from jax.experimental import pallas as pl
from jax.experimental.pallas import tpu as pltpu

B = 4
S = 2048
D = 768
P = 1000
TOP_K = 4
BLK = 256
SIZES = (64, 256, 256, 256, 256, 256, 256, 256, 192)
OFFS = tuple(sum(SIZES[:j]) for j in range(len(SIZES)))
N_BLK = len(SIZES)
OUT_DEPTH = 3


def _body(x_hbm, prompt_hbm, out_hbm, sim_ref, idx_ref, rsim_ref,
          xbuf, sbuf, pbuf, acc, first4, tbuf, head,
          in_sems, out_sems, p_sem, head_sem, tail_sem):

    def in_copy(blk):
        off, sz = OFFS[blk], SIZES[blk]
        return pltpu.make_async_copy(
            x_hbm.at[:, pl.ds(off, sz), :],
            xbuf.at[blk, :, pl.ds(0, sz), :],
            in_sems.at[blk])

    def out_copy(blk):
        off, sz = OFFS[blk], SIZES[blk]
        return pltpu.make_async_copy(
            sbuf.at[blk % OUT_DEPTH, :, pl.ds(0, sz), :],
            out_hbm.at[:, pl.ds(off, sz), :],
            out_sems.at[blk % OUT_DEPTH])

    for blk in range(N_BLK):
        in_copy(blk).start()
    pltpu.make_async_copy(prompt_hbm, pbuf, p_sem).start()

    acc_v = jnp.zeros((B, D), jnp.float32)
    tail = None
    for blk in range(N_BLK):
        sz = SIZES[blk]
        in_copy(blk).wait()
        v = xbuf[blk, :, :sz, :]
        acc_v = acc_v + jnp.sum(v, axis=1)
        if blk == 0:
            first4[...] = v[:, :TOP_K, :]
            shifted = jnp.concatenate(
                [v[:, :TOP_K, :], v[:, :sz - TOP_K, :]], axis=1)
        else:
            shifted = jnp.concatenate([tail, v[:, :sz - TOP_K, :]], axis=1)
        tail = v[:, sz - TOP_K:, :]
        if blk >= OUT_DEPTH:
            out_copy(blk - OUT_DEPTH).wait()
        sbuf[blk % OUT_DEPTH, :, :sz, :] = shifted
        out_copy(blk).start()
    acc[...] = acc_v

    mean = acc_v * (1.0 / S)
    xn = mean * jax.lax.rsqrt(
        jnp.maximum(jnp.sum(mean * mean, axis=1, keepdims=True), 1e-12))
    pltpu.make_async_copy(prompt_hbm, pbuf, p_sem).wait()
    p = pbuf[...]
    pn = p * jax.lax.rsqrt(
        jnp.maximum(jnp.sum(p * p, axis=1, keepdims=True), 1e-12))
    sim = jax.lax.dot_general(
        xn, pn, (((1,), (1,)), ((), ())),
        preferred_element_type=jnp.float32)
    sim_ref[...] = sim

    iota = jax.lax.broadcasted_iota(jnp.int32, (B, P), 1)
    s = sim
    total = jnp.float32(0.0)
    idx_cols = []
    bp_cols = []
    for k in range(TOP_K):
        m = jnp.max(s, axis=1, keepdims=True)
        eq = s == m
        ik = jnp.min(jnp.where(eq, iota, P), axis=1)
        sel = iota == ik[:, None]
        idx_cols.append(ik)
        total += jnp.sum(m)
        bp_cols.append(jax.lax.dot_general(
            sel.astype(jnp.float32), p, (((1,), (0,)), ((), ())),
            preferred_element_type=jnp.float32))
        s = jnp.where(sel, -jnp.inf, s)
    idx_ref[...] = jnp.stack(idx_cols, axis=1)
    rsim_ref[...] = jnp.reshape(total * (1.0 / B), (1, 1))

    head[...] = jnp.concatenate(
        [jnp.stack(bp_cols, axis=1), first4[...]], axis=1)
    hcopy = pltpu.make_async_copy(
        head, out_hbm.at[:, pl.ds(0, 2 * TOP_K), :], head_sem)
    hcopy.start()
    tbuf[...] = tail
    tcopy = pltpu.make_async_copy(
        tbuf, out_hbm.at[:, pl.ds(S, TOP_K), :], tail_sem)
    tcopy.start()
    for blk in range(N_BLK - OUT_DEPTH, N_BLK):
        out_copy(blk).wait()
    hcopy.wait()
    tcopy.wait()


def kernel(x_embed, prompt):
    out_shapes = (
        jax.ShapeDtypeStruct((B, TOP_K + S, D), jnp.float32),
        jax.ShapeDtypeStruct((B, P), jnp.float32),
        jax.ShapeDtypeStruct((B, TOP_K), jnp.int32),
        jax.ShapeDtypeStruct((1, 1), jnp.float32),
    )
    prompted, sim, idx, rsim = pl.pallas_call(
        _body,
        in_specs=[
            pl.BlockSpec(memory_space=pl.MemorySpace.ANY),
            pl.BlockSpec(memory_space=pl.MemorySpace.ANY),
        ],
        out_specs=(
            pl.BlockSpec(memory_space=pl.MemorySpace.ANY),
            pl.BlockSpec((B, P), lambda: (0, 0)),
            pl.BlockSpec((B, TOP_K), lambda: (0, 0)),
            pl.BlockSpec((1, 1), lambda: (0, 0)),
        ),
        out_shape=out_shapes,
        scratch_shapes=[
            pltpu.VMEM((N_BLK, B, BLK, D), jnp.float32),
            pltpu.VMEM((OUT_DEPTH, B, BLK, D), jnp.float32),
            pltpu.VMEM((P, D), jnp.float32),
            pltpu.VMEM((B, D), jnp.float32),
            pltpu.VMEM((B, TOP_K, D), jnp.float32),
            pltpu.VMEM((B, TOP_K, D), jnp.float32),
            pltpu.VMEM((B, 2 * TOP_K, D), jnp.float32),
            pltpu.SemaphoreType.DMA((N_BLK,)),
            pltpu.SemaphoreType.DMA((OUT_DEPTH,)),
            pltpu.SemaphoreType.DMA,
            pltpu.SemaphoreType.DMA,
            pltpu.SemaphoreType.DMA,
        ],
    )(x_embed, prompt)
    return prompted, rsim[0, 0], sim, idx
